# baseline (speedup 1.0000x reference)
import functools

import jax
import jax.numpy as jnp
from jax import lax
from jax.experimental import pallas as pl
from jax.experimental.pallas import tpu as pltpu
from jax.experimental.pallas import tpu_sc as plsc

NUM_VOCAB = 1000000
NUM_EMBED = 64
NUM_TOKEN = 200
BATCH = 1024

NW = 32
B_TOTAL = BATCH * NUM_TOKEN
TOK_W = B_TOTAL // NW
CHUNK = 128
NCHUNK = TOK_W // CHUNK
LANES = 16
C_PER_ROW = NUM_EMBED // LANES
PADD = 128

IDX_ROWS = B_TOTAL // PADD


B_PER_W = BATCH // NW


def _emb_kernel(tok_hbm, table_hbm, pos_hbm, out_hbm, idxt_v, idx_v, pos2_v,
                gbuf0, gbuf1, rows0, rows1, gsem0, gsem1, ssem0, ssem1):
  wid = lax.axis_index("s") * 2 + lax.axis_index("c")
  base = wid * TOK_W

  pltpu.sync_copy(tok_hbm.at[pl.ds(0, NUM_TOKEN), pl.ds(wid * B_PER_W,
                                                        B_PER_W)], idxt_v)
  pltpu.sync_copy(pos_hbm, pos2_v.at[pl.ds(0, NUM_TOKEN)])
  pltpu.sync_copy(pos_hbm, pos2_v.at[pl.ds(NUM_TOKEN, NUM_TOKEN)])

  t_offs = [LANES * k for k in range(12)] + [NUM_TOKEN - LANES]

  def tconv_body(p, carry):
    pv = jnp.full((LANES,), 0, jnp.int32) + p
    for off in t_offs:
      tv = off + lax.iota(jnp.int32, LANES)
      v = plsc.load_gather(idxt_v, [tv, pv])
      idx_v[pl.ds(pl.multiple_of(p * NUM_TOKEN + off, 8), LANES)] = v
    return carry

  lax.fori_loop(0, B_PER_W, tconv_body, 0)

  gbufs = (gbuf0, gbuf1)
  rows = (rows0, rows1)
  gsems = (gsem0, gsem1)
  ssems = (ssem0, ssem1)

  def gather(j, g):
    return pltpu.async_copy(table_hbm.at[idx_v.at[pl.ds(j * CHUNK, CHUNK)]],
                            gbufs[g], gsems[g])

  def gwait(g):
    pltpu.make_async_copy(table_hbm.at[idx_v.at[pl.ds(0, CHUNK)]], gbufs[g],
                          gsems[g]).wait()

  def swait(g):
    pltpu.make_async_copy(rows[g], out_hbm.at[pl.ds(0, CHUNK)],
                          ssems[g]).wait()

  gather(0, 0)

  def pair_body(j2, carry):
    for g in range(2):
      j = j2 * 2 + g
      gwait(g)

      @pl.when(j + 1 < NCHUNK)
      def _():
        gather(j + 1, 1 - g)

      @pl.when(j >= 2)
      def _():
        swait(g)

      phase = lax.rem(j * CHUNK, NUM_TOKEN)

      def row_body(r, c2):
        for c in range(C_PER_ROW):
          sl = pl.ds(c * LANES, LANES)
          rows[g][r, sl] = gbufs[g][r, sl] + pos2_v[phase + r, sl]
        return c2

      lax.fori_loop(0, CHUNK, row_body, 0)
      pltpu.async_copy(rows[g],
                       out_hbm.at[pl.ds(base + j * CHUNK, CHUNK)], ssems[g])
    return carry

  lax.fori_loop(0, NCHUNK // 2, pair_body, 0)
  swait(0)
  swait(1)


@jax.jit
def _emb(tokens_t, table, positionembed):
  mesh = plsc.VectorSubcoreMesh(core_axis_name="c", subcore_axis_name="s")
  run = functools.partial(
      pl.kernel,
      mesh=mesh,
      compiler_params=pltpu.CompilerParams(use_tc_tiling_on_sc=False,
                                           needs_layout_passes=False),
      out_type=jax.ShapeDtypeStruct((B_TOTAL, PADD), jnp.float32),
      scratch_types=[
          pltpu.VMEM((NUM_TOKEN, B_PER_W), jnp.int32),
          pltpu.VMEM((TOK_W,), jnp.int32),
          pltpu.VMEM((2 * NUM_TOKEN, NUM_EMBED), jnp.float32),
          pltpu.VMEM((CHUNK, NUM_EMBED), jnp.float32),
          pltpu.VMEM((CHUNK, NUM_EMBED), jnp.float32),
          pltpu.VMEM((CHUNK, PADD), jnp.float32),
          pltpu.VMEM((CHUNK, PADD), jnp.float32),
          pltpu.SemaphoreType.DMA,
          pltpu.SemaphoreType.DMA,
          pltpu.SemaphoreType.DMA,
          pltpu.SemaphoreType.DMA,
      ],
  )(_emb_kernel)
  return run(tokens_t, table, positionembed)


def _unpack_kernel(x_ref, o_ref):
  x = x_ref[...]
  o_ref[...] = x[:, :NUM_EMBED].reshape(o_ref.shape)


@jax.jit
def _unpack(padded):
  return pl.pallas_call(
      _unpack_kernel,
      grid=(BATCH // 8,),
      in_specs=[pl.BlockSpec((8 * NUM_TOKEN, PADD), lambda i: (i, 0))],
      out_specs=pl.BlockSpec((8, NUM_TOKEN, NUM_EMBED), lambda i: (i, 0, 0)),
      out_shape=jax.ShapeDtypeStruct((BATCH, NUM_TOKEN, NUM_EMBED),
                                     jnp.float32),
  )(padded)


def kernel(tokens, table, positionembed):
  tokens_t = tokens.astype(jnp.int32).T
  padded = _emb(tokens_t, table, positionembed)
  return _unpack(padded)
